# baseline (speedup 1.0000x reference)
import functools

import jax
import jax.numpy as jnp
from jax import lax
from jax.experimental import pallas as pl
from jax.experimental.pallas import tpu as pltpu
from jax.experimental.pallas import tpu_sc as plsc

F32 = jnp.float32


def _pick_block(n, cap=1024):
    for c in (1024, 1000, 800, 640, 512, 400, 256, 250, 200, 128, 100, 80, 64, 40, 32, 16, 8):
        if c <= cap and n % c == 0:
            return c
    return n


def _mlp2(x, w1t, b1, w2t, b2):
    n, din = x.shape
    dmid = w1t.shape[1]
    dout = w2t.shape[1]
    br = _pick_block(n)

    def body(x_ref, w1_ref, b1_ref, w2_ref, b2_ref, o_ref):
        z = jnp.maximum(
            jnp.dot(x_ref[...], w1_ref[...], preferred_element_type=F32) + b1_ref[...], 0.0)
        o_ref[...] = jnp.dot(z, w2_ref[...], preferred_element_type=F32) + b2_ref[...]

    return pl.pallas_call(
        body,
        grid=(n // br,),
        in_specs=[
            pl.BlockSpec((br, din), lambda i: (i, 0)),
            pl.BlockSpec((din, dmid), lambda i: (0, 0)),
            pl.BlockSpec((1, dmid), lambda i: (0, 0)),
            pl.BlockSpec((dmid, dout), lambda i: (0, 0)),
            pl.BlockSpec((1, dout), lambda i: (0, 0)),
        ],
        out_specs=pl.BlockSpec((br, dout), lambda i: (i, 0)),
        out_shape=jax.ShapeDtypeStruct((n, dout), F32),
    )(x, w1t, b1, w2t, b2)


def _gin_res_stats(h, a0, a1, w1t, b1, w2t, b2):
    n, d = h.shape
    br = _pick_block(n)
    ng = n // br

    def body(h_ref, a0_ref, a1_ref, w1_ref, b1_ref, w2_ref, b2_ref, t_ref, st_ref):
        i = pl.program_id(0)
        hh = h_ref[...]
        loc0 = hh + a0_ref[...] + a1_ref[...]
        z = jnp.maximum(jnp.dot(loc0, w1_ref[...], preferred_element_type=F32) + b1_ref[...], 0.0)
        t = jnp.dot(z, w2_ref[...], preferred_element_type=F32) + b2_ref[...] + hh
        t_ref[...] = t

        @pl.when(i == 0)
        def _():
            st_ref[...] = jnp.zeros_like(st_ref)

        st_ref[0:1, :] += jnp.sum(t, axis=0, keepdims=True)
        st_ref[1:2, :] += jnp.sum(t * t, axis=0, keepdims=True)

    return pl.pallas_call(
        body,
        grid=(ng,),
        in_specs=[
            pl.BlockSpec((br, d), lambda i: (i, 0)),
            pl.BlockSpec((br, d), lambda i: (i, 0)),
            pl.BlockSpec((br, d), lambda i: (i, 0)),
            pl.BlockSpec((d, d), lambda i: (0, 0)),
            pl.BlockSpec((1, d), lambda i: (0, 0)),
            pl.BlockSpec((d, d), lambda i: (0, 0)),
            pl.BlockSpec((1, d), lambda i: (0, 0)),
        ],
        out_specs=[
            pl.BlockSpec((br, d), lambda i: (i, 0)),
            pl.BlockSpec((2, d), lambda i: (0, 0)),
        ],
        out_shape=[
            jax.ShapeDtypeStruct((n, d), F32),
            jax.ShapeDtypeStruct((2, d), F32),
        ],
    )(h, a0, a1, w1t, b1, w2t, b2)


def _matmul_bias(x, wt, b, bc=512):
    n, din = x.shape
    dout = wt.shape[1]
    br = _pick_block(n, cap=512)

    def body(x_ref, w_ref, b_ref, o_ref):
        o_ref[...] = jnp.dot(x_ref[...], w_ref[...], preferred_element_type=F32) + b_ref[...]

    return pl.pallas_call(
        body,
        grid=(n // br, dout // bc),
        in_specs=[
            pl.BlockSpec((br, din), lambda i, j: (i, 0)),
            pl.BlockSpec((din, bc), lambda i, j: (0, j)),
            pl.BlockSpec((1, bc), lambda i, j: (0, j)),
        ],
        out_specs=pl.BlockSpec((br, bc), lambda i, j: (i, j)),
        out_shape=jax.ShapeDtypeStruct((n, dout), F32),
    )(x, wt, b)


def _flash_attn(qkv, nheads, nvalid, dh):
    nq = qkv.shape[0]
    bq = 512
    bk = 512
    nqb = nq // bq
    nkb = nq // bk
    scale = 1.0 / float(dh) ** 0.5

    def body(q_ref, k_ref, v_ref, o_ref, m_scr, l_scr, acc_scr):
        ki = pl.program_id(2)
        nk = pl.num_programs(2)

        @pl.when(ki == 0)
        def _():
            m_scr[...] = jnp.full_like(m_scr, -1e30)
            l_scr[...] = jnp.zeros_like(l_scr)
            acc_scr[...] = jnp.zeros_like(acc_scr)

        q = q_ref[...].astype(jnp.bfloat16)
        k = k_ref[...].astype(jnp.bfloat16)
        s = lax.dot_general(q, k, (((1,), (1,)), ((), ())),
                            preferred_element_type=F32) * scale
        col = ki * bk + lax.broadcasted_iota(jnp.int32, (bq, bk), 1)
        s = jnp.where(col < nvalid, s, -1e30)
        m_prev = m_scr[:, 0:1]
        m_cur = jnp.max(s, axis=1, keepdims=True)
        m_next = jnp.maximum(m_prev, m_cur)
        alpha = jnp.exp(m_prev - m_next)
        p = jnp.exp(s - m_next)
        l_next = l_scr[:, 0:1] * alpha + jnp.sum(p, axis=1, keepdims=True)
        acc_scr[...] = acc_scr[...] * alpha + jnp.dot(
            p.astype(jnp.bfloat16), v_ref[...].astype(jnp.bfloat16),
            preferred_element_type=F32)
        m_scr[...] = jnp.broadcast_to(m_next, m_scr.shape)
        l_scr[...] = jnp.broadcast_to(l_next, l_scr.shape)

        @pl.when(ki == nk - 1)
        def _():
            o_ref[...] = acc_scr[...] / l_scr[:, 0:1]

    return pl.pallas_call(
        body,
        grid=(nheads, nqb, nkb),
        in_specs=[
            pl.BlockSpec((bq, 128), lambda h, qi, ki: (qi, h)),
            pl.BlockSpec((bk, 128), lambda h, qi, ki: (ki, nheads + h)),
            pl.BlockSpec((bk, 128), lambda h, qi, ki: (ki, 2 * nheads + h)),
        ],
        out_specs=pl.BlockSpec((bq, 128), lambda h, qi, ki: (qi, h)),
        out_shape=jax.ShapeDtypeStruct((nq, nheads * 128), F32),
        scratch_shapes=[
            pltpu.VMEM((bq, 128), F32),
            pltpu.VMEM((bq, 128), F32),
            pltpu.VMEM((bq, 128), F32),
        ],
    )(qkv, qkv, qkv)


def _lin_res_stats(o, wt, b, h):
    n, din = o.shape
    d = h.shape[1]
    br = _pick_block(n, cap=1000)
    ng = n // br

    def body(o_ref, w_ref, b_ref, h_ref, t_ref, st_ref):
        i = pl.program_id(0)
        t = jnp.dot(o_ref[...], w_ref[...], preferred_element_type=F32) + b_ref[...] + h_ref[...]
        t_ref[...] = t

        @pl.when(i == 0)
        def _():
            st_ref[...] = jnp.zeros_like(st_ref)

        st_ref[0:1, :] += jnp.sum(t, axis=0, keepdims=True)
        st_ref[1:2, :] += jnp.sum(t * t, axis=0, keepdims=True)

    return pl.pallas_call(
        body,
        grid=(ng,),
        in_specs=[
            pl.BlockSpec((br, din), lambda i: (i, 0)),
            pl.BlockSpec((din, d), lambda i: (0, 0)),
            pl.BlockSpec((1, d), lambda i: (0, 0)),
            pl.BlockSpec((br, d), lambda i: (i, 0)),
        ],
        out_specs=[
            pl.BlockSpec((br, d), lambda i: (i, 0)),
            pl.BlockSpec((2, d), lambda i: (0, 0)),
        ],
        out_shape=[
            jax.ShapeDtypeStruct((n, d), F32),
            jax.ShapeDtypeStruct((2, d), F32),
        ],
    )(o, wt, b, h)


def _combine_mlp_stats(t1, st1, t2, st2, g1, c1, g2, c2, m1t, mb1, m2t, mb2):
    n, d = t1.shape
    dmid = m1t.shape[1]
    br = _pick_block(n, cap=1000)
    ng = n // br
    nf = float(n)

    def body(t1_ref, s1_ref, t2_ref, s2_ref, g1_ref, c1_ref, g2_ref, c2_ref,
             w1_ref, b1_ref, w2_ref, b2_ref, t3_ref, st_ref):
        i = pl.program_id(0)
        mu1 = s1_ref[0:1, :] / nf
        va1 = s1_ref[1:2, :] / nf - mu1 * mu1
        h1 = g1_ref[...] * (t1_ref[...] - mu1) / jnp.sqrt(va1 + 1e-5) + c1_ref[...]
        mu2 = s2_ref[0:1, :] / nf
        va2 = s2_ref[1:2, :] / nf - mu2 * mu2
        h2 = g2_ref[...] * (t2_ref[...] - mu2) / jnp.sqrt(va2 + 1e-5) + c2_ref[...]
        op = h1 + h2
        z = jnp.maximum(jnp.dot(op, w1_ref[...], preferred_element_type=F32) + b1_ref[...], 0.0)
        t3 = op + jnp.dot(z, w2_ref[...], preferred_element_type=F32) + b2_ref[...]
        t3_ref[...] = t3

        @pl.when(i == 0)
        def _():
            st_ref[...] = jnp.zeros_like(st_ref)

        st_ref[0:1, :] += jnp.sum(t3, axis=0, keepdims=True)
        st_ref[1:2, :] += jnp.sum(t3 * t3, axis=0, keepdims=True)

    full = lambda shape: pl.BlockSpec(shape, lambda i: (0, 0))
    rows = pl.BlockSpec((br, d), lambda i: (i, 0))
    return pl.pallas_call(
        body,
        grid=(ng,),
        in_specs=[
            rows, full((2, d)), rows, full((2, d)),
            full((1, d)), full((1, d)), full((1, d)), full((1, d)),
            full((d, dmid)), full((1, dmid)), full((dmid, d)), full((1, d)),
        ],
        out_specs=[
            pl.BlockSpec((br, d), lambda i: (i, 0)),
            pl.BlockSpec((2, d), lambda i: (0, 0)),
        ],
        out_shape=[
            jax.ShapeDtypeStruct((n, d), F32),
            jax.ShapeDtypeStruct((2, d), F32),
        ],
    )(t1, st1, t2, st2, g1, c1, g2, c2, m1t, mb1, m2t, mb2)


def _final_bn_dec(t3, st3, g3, c3, decwt):
    n, d = t3.shape
    br = _pick_block(n, cap=1000)
    nf = float(n)

    def body(t_ref, s_ref, g_ref, c_ref, w_ref, o_ref, q_ref):
        mu = s_ref[0:1, :] / nf
        va = s_ref[1:2, :] / nf - mu * mu
        out = g_ref[...] * (t_ref[...] - mu) / jnp.sqrt(va + 1e-5) + c_ref[...]
        o_ref[...] = out
        q_ref[...] = jnp.dot(out, w_ref[...], preferred_element_type=F32)

    full = lambda shape: pl.BlockSpec(shape, lambda i: (0, 0))
    return pl.pallas_call(
        body,
        grid=(n // br,),
        in_specs=[
            pl.BlockSpec((br, d), lambda i: (i, 0)),
            full((2, d)), full((1, d)), full((1, d)), full((d, d)),
        ],
        out_specs=[
            pl.BlockSpec((br, d), lambda i: (i, 0)),
            pl.BlockSpec((br, d), lambda i: (i, 0)),
        ],
        out_shape=[
            jax.ShapeDtypeStruct((n, d), F32),
            jax.ShapeDtypeStruct((n, d), F32),
        ],
    )(t3, st3, g3, c3, decwt)


_NC = 2
_NS = 16
_NW = _NC * _NS


def _lane_gather(v, idx):
    dnums = lax.GatherDimensionNumbers(
        offset_dims=(), collapsed_slice_dims=(0,), start_index_map=(0,))
    return lax.gather(v, idx[:, None], dnums, (1,),
                      mode=lax.GatherScatterMode.PROMISE_IN_BOUNDS)


def _sc_message(src, dst, h, ee, zeros_init):
    e = src.shape[0]
    hd = h.shape[1]
    racc = zeros_init.shape[0]
    epw = e // _NW
    c_sz = 80
    nch = epw // c_sz
    rpt = racc // _NS
    mesh = plsc.VectorSubcoreMesh(core_axis_name="c", subcore_axis_name="s")

    @functools.partial(
        pl.kernel,
        out_type=jax.ShapeDtypeStruct((_NC * racc, hd), F32),
        mesh=mesh,
        scratch_types=[
            pltpu.VMEM((c_sz,), jnp.int32),
            pltpu.VMEM((c_sz,), jnp.int32),
            pltpu.VMEM((c_sz, hd), F32),
            pltpu.VMEM((c_sz, hd), F32),
            pltpu.VMEM_SHARED((racc, hd), F32),
            pltpu.SemaphoreType.DMA,
        ],
    )
    def k(src_hbm, dst_hbm, h_hbm, ee_hbm, z_hbm, out_hbm,
          src_v, dst_v, hrow_v, ee_v, acc_sh, sem):
        c = lax.axis_index("c")
        s = lax.axis_index("s")
        wid = c * _NS + s
        pltpu.sync_copy(z_hbm.at[pl.ds(s * rpt, rpt)], acc_sh.at[pl.ds(s * rpt, rpt)])
        plsc.subcore_barrier()

        def chunk(i, carry):
            base = wid * epw + i * c_sz
            pltpu.sync_copy(src_hbm.at[pl.ds(base, c_sz)], src_v)
            pltpu.sync_copy(dst_hbm.at[pl.ds(base, c_sz)], dst_v)
            pltpu.async_copy(h_hbm.at[src_v], hrow_v, sem).wait()
            pltpu.sync_copy(ee_hbm.at[pl.ds(base, c_sz)], ee_v)

            def row(r, carry2):
                for j in range(hd // 16):
                    sl = pl.ds(j * 16, 16)
                    hrow_v[r, sl] = jnp.maximum(hrow_v[r, sl] + ee_v[r, sl], 0.0)
                return carry2

            lax.fori_loop(0, c_sz, row, 0)
            pltpu.sync_copy(hrow_v, acc_sh.at[dst_v], add=True)
            return carry

        lax.fori_loop(0, nch, chunk, 0)
        plsc.subcore_barrier()
        pltpu.sync_copy(acc_sh.at[pl.ds(s * rpt, rpt)],
                        out_hbm.at[pl.ds(c * racc + s * rpt, rpt)])

    return k(src, dst, h, ee, zeros_init)


def _sc_decode(out3, q2, oi, di):
    pp = oi.shape[0]
    hd = out3.shape[1]
    ppw = pp // _NW
    cd = 128
    nch = ppw // cd
    mesh = plsc.VectorSubcoreMesh(core_axis_name="c", subcore_axis_name="s")

    @functools.partial(
        pl.kernel,
        out_type=jax.ShapeDtypeStruct((pp,), F32),
        mesh=mesh,
        scratch_types=[
            pltpu.VMEM((cd,), jnp.int32),
            pltpu.VMEM((cd,), jnp.int32),
            pltpu.VMEM((cd, hd), F32),
            pltpu.VMEM((cd, hd), F32),
            pltpu.VMEM((cd,), F32),
            pltpu.SemaphoreType.DMA,
        ],
    )
    def k(o_hbm, q_hbm, oi_hbm, di_hbm, res_hbm, oi_v, di_v, oe_v, de_v, res_v, sem):
        c = lax.axis_index("c")
        s = lax.axis_index("s")
        wid = c * _NS + s
        lane = lax.broadcasted_iota(jnp.int32, (16,), 0)

        def chunk(i, carry):
            base = wid * ppw + i * cd
            pltpu.sync_copy(oi_hbm.at[pl.ds(base, cd)], oi_v)
            pltpu.sync_copy(di_hbm.at[pl.ds(base, cd)], di_v)
            pltpu.async_copy(o_hbm.at[oi_v], oe_v, sem).wait()
            pltpu.async_copy(q_hbm.at[di_v], de_v, sem).wait()

            def grp(g, carry2):
                vec = jnp.zeros((16,), F32)
                for jj in range(16):
                    r = g * 16 + jj
                    acc = jnp.zeros((16,), F32)
                    for j in range(hd // 16):
                        sl = pl.ds(j * 16, 16)
                        acc = acc + oe_v[r, sl] * de_v[r, sl]
                    for kk in (1, 2, 4, 8):
                        acc = acc + _lane_gather(acc, lane ^ kk)
                    vec = jnp.where(lane == jj, acc, vec)
                res_v[pl.ds(g * 16, 16)] = vec
                return carry2

            lax.fori_loop(0, cd // 16, grp, 0)
            pltpu.sync_copy(res_v, res_hbm.at[pl.ds(base, cd)])
            return carry

        lax.fori_loop(0, nch, chunk, 0)

    return k(out3, q2, oi, di)


def kernel(x, edge_attr, params, edge_index, origin_idx, dest_idx):
    p = params
    n, idim = x.shape
    e = edge_attr.shape[0]
    hd = p["np2_W"].shape[0]
    nh = 4
    dh = hd // nh
    npairs = origin_idx.shape[0]

    r2 = lambda v: v.reshape(1, -1)

    h = _mlp2(x, p["np1_W"].T, r2(p["np1_b"]), p["np2_W"].T, r2(p["np2_b"]))
    ee = _mlp2(edge_attr, p["ep1_W"].T, r2(p["ep1_b"]), p["ep2_W"].T, r2(p["ep2_b"]))

    racc = 10240
    zinit = jnp.zeros((racc, hd), F32)
    parts = _sc_message(edge_index[0], edge_index[1], h, ee, zinit)
    a0 = lax.slice(parts, (0, 0), (n, hd))
    a1 = lax.slice(parts, (racc, 0), (racc + n, hd))

    t1, st1 = _gin_res_stats(h, a0, a1, p["gin1_W"].T, r2(p["gin1_b"]),
                             p["gin2_W"].T, r2(p["gin2_b"]))

    bq = 512
    nqp = ((n + bq - 1) // bq) * bq
    h_pad = jnp.pad(h, ((0, nqp - n), (0, 0)))
    win = p["attn_in_W"]
    bin_ = p["attn_in_b"]
    wpad = jnp.zeros((hd, 3 * nh * 128), F32)
    bpad = jnp.zeros((3 * nh * 128,), F32)
    for part in range(3):
        for hh in range(nh):
            src_lo = part * hd + hh * dh
            dst_lo = (part * nh + hh) * 128
            wpad = wpad.at[:, dst_lo:dst_lo + dh].set(win[src_lo:src_lo + dh, :].T)
            bpad = bpad.at[dst_lo:dst_lo + dh].set(bin_[src_lo:src_lo + dh])
    qkv = _matmul_bias(h_pad, wpad, r2(bpad))

    o_all = _flash_attn(qkv, nh, n, dh)

    wo = p["attn_out_W"]
    wo_pad = jnp.zeros((nh * 128, hd), F32)
    for hh in range(nh):
        wo_pad = wo_pad.at[hh * 128:hh * 128 + dh, :].set(wo[:, hh * dh:(hh + 1) * dh].T)
    t2, st2 = _lin_res_stats(lax.slice(o_all, (0, 0), (n, nh * 128)),
                             wo_pad, r2(p["attn_out_b"]), h)

    t3, st3 = _combine_mlp_stats(
        t1, st1, t2, st2,
        r2(p["n1_g"]), r2(p["n1_b"]), r2(p["n2_g"]), r2(p["n2_b"]),
        p["mlp1_W"].T, r2(p["mlp1_b"]), p["mlp2_W"].T, r2(p["mlp2_b"]))

    out3, q2 = _final_bn_dec(t3, st3, r2(p["n3_g"]), r2(p["n3_b"]), p["dec_W"].T)

    ppad = ((npairs + 4096 - 1) // 4096) * 4096
    oi = jnp.pad(origin_idx, (0, ppad - npairs))
    di = jnp.pad(dest_idx, (0, ppad - npairs))
    res = _sc_decode(out3, q2, oi, di)
    return lax.slice(res, (0,), (npairs,))


# speedup vs baseline: 1.8393x; 1.8393x over previous
import functools

import jax
import jax.numpy as jnp
from jax import lax
from jax.experimental import pallas as pl
from jax.experimental.pallas import tpu as pltpu
from jax.experimental.pallas import tpu_sc as plsc

F32 = jnp.float32


def _pick_block(n, cap=1024):
    for c in (1024, 1000, 800, 640, 512, 400, 256, 250, 200, 128, 100, 80, 64, 40, 32, 16, 8):
        if c <= cap and n % c == 0:
            return c
    return n


def _mlp2(x, w1t, b1, w2t, b2):
    n, din = x.shape
    dmid = w1t.shape[1]
    dout = w2t.shape[1]
    br = _pick_block(n)

    def body(x_ref, w1_ref, b1_ref, w2_ref, b2_ref, o_ref):
        z = jnp.maximum(
            jnp.dot(x_ref[...], w1_ref[...], preferred_element_type=F32) + b1_ref[...], 0.0)
        o_ref[...] = jnp.dot(z, w2_ref[...], preferred_element_type=F32) + b2_ref[...]

    return pl.pallas_call(
        body,
        grid=(n // br,),
        in_specs=[
            pl.BlockSpec((br, din), lambda i: (i, 0)),
            pl.BlockSpec((din, dmid), lambda i: (0, 0)),
            pl.BlockSpec((1, dmid), lambda i: (0, 0)),
            pl.BlockSpec((dmid, dout), lambda i: (0, 0)),
            pl.BlockSpec((1, dout), lambda i: (0, 0)),
        ],
        out_specs=pl.BlockSpec((br, dout), lambda i: (i, 0)),
        out_shape=jax.ShapeDtypeStruct((n, dout), F32),
    )(x, w1t, b1, w2t, b2)


def _gin_res_stats(h, a0, a1, w1t, b1, w2t, b2):
    n, d = h.shape
    br = _pick_block(n)
    ng = n // br

    def body(h_ref, a0_ref, a1_ref, w1_ref, b1_ref, w2_ref, b2_ref, t_ref, st_ref):
        i = pl.program_id(0)
        hh = h_ref[...]
        loc0 = hh + a0_ref[...] + a1_ref[...]
        z = jnp.maximum(jnp.dot(loc0, w1_ref[...], preferred_element_type=F32) + b1_ref[...], 0.0)
        t = jnp.dot(z, w2_ref[...], preferred_element_type=F32) + b2_ref[...] + hh
        t_ref[...] = t

        @pl.when(i == 0)
        def _():
            st_ref[...] = jnp.zeros_like(st_ref)

        st_ref[0:1, :] += jnp.sum(t, axis=0, keepdims=True)
        st_ref[1:2, :] += jnp.sum(t * t, axis=0, keepdims=True)

    return pl.pallas_call(
        body,
        grid=(ng,),
        in_specs=[
            pl.BlockSpec((br, d), lambda i: (i, 0)),
            pl.BlockSpec((br, d), lambda i: (i, 0)),
            pl.BlockSpec((br, d), lambda i: (i, 0)),
            pl.BlockSpec((d, d), lambda i: (0, 0)),
            pl.BlockSpec((1, d), lambda i: (0, 0)),
            pl.BlockSpec((d, d), lambda i: (0, 0)),
            pl.BlockSpec((1, d), lambda i: (0, 0)),
        ],
        out_specs=[
            pl.BlockSpec((br, d), lambda i: (i, 0)),
            pl.BlockSpec((2, d), lambda i: (0, 0)),
        ],
        out_shape=[
            jax.ShapeDtypeStruct((n, d), F32),
            jax.ShapeDtypeStruct((2, d), F32),
        ],
    )(h, a0, a1, w1t, b1, w2t, b2)


def _matmul_bias(x, wt, b, bc=512):
    n, din = x.shape
    dout = wt.shape[1]
    br = _pick_block(n, cap=512)

    def body(x_ref, w_ref, b_ref, o_ref):
        o_ref[...] = jnp.dot(x_ref[...], w_ref[...], preferred_element_type=F32) + b_ref[...]

    return pl.pallas_call(
        body,
        grid=(n // br, dout // bc),
        in_specs=[
            pl.BlockSpec((br, din), lambda i, j: (i, 0)),
            pl.BlockSpec((din, bc), lambda i, j: (0, j)),
            pl.BlockSpec((1, bc), lambda i, j: (0, j)),
        ],
        out_specs=pl.BlockSpec((br, bc), lambda i, j: (i, j)),
        out_shape=jax.ShapeDtypeStruct((n, dout), F32),
    )(x, wt, b)


def _attn_direct(qkv, nheads, dh):
    nq = qkv.shape[0]
    bq = 400
    nqb = nq // bq
    scale = 1.0 / float(dh) ** 0.5

    def body(q_ref, k_ref, v_ref, o_ref):
        s = lax.dot_general(q_ref[...], k_ref[...], (((1,), (1,)), ((), ())),
                            preferred_element_type=F32) * scale
        m = jnp.max(s, axis=1, keepdims=True)
        p = jnp.exp(s - m)
        l = jnp.sum(p, axis=1, keepdims=True)
        o_ref[...] = jnp.dot(p, v_ref[...], preferred_element_type=F32) / l

    return pl.pallas_call(
        body,
        grid=(nheads, nqb),
        in_specs=[
            pl.BlockSpec((bq, 128), lambda h, qi: (qi, h)),
            pl.BlockSpec((nq, 128), lambda h, qi: (0, nheads + h)),
            pl.BlockSpec((nq, 128), lambda h, qi: (0, 2 * nheads + h)),
        ],
        out_specs=pl.BlockSpec((bq, 128), lambda h, qi: (qi, h)),
        out_shape=jax.ShapeDtypeStruct((nq, nheads * 128), F32),
    )(qkv, qkv, qkv)


def _lin_res_stats(o, wt, b, h):
    n, din = o.shape
    d = h.shape[1]
    br = _pick_block(n, cap=1000)
    ng = n // br

    def body(o_ref, w_ref, b_ref, h_ref, t_ref, st_ref):
        i = pl.program_id(0)
        t = jnp.dot(o_ref[...], w_ref[...], preferred_element_type=F32) + b_ref[...] + h_ref[...]
        t_ref[...] = t

        @pl.when(i == 0)
        def _():
            st_ref[...] = jnp.zeros_like(st_ref)

        st_ref[0:1, :] += jnp.sum(t, axis=0, keepdims=True)
        st_ref[1:2, :] += jnp.sum(t * t, axis=0, keepdims=True)

    return pl.pallas_call(
        body,
        grid=(ng,),
        in_specs=[
            pl.BlockSpec((br, din), lambda i: (i, 0)),
            pl.BlockSpec((din, d), lambda i: (0, 0)),
            pl.BlockSpec((1, d), lambda i: (0, 0)),
            pl.BlockSpec((br, d), lambda i: (i, 0)),
        ],
        out_specs=[
            pl.BlockSpec((br, d), lambda i: (i, 0)),
            pl.BlockSpec((2, d), lambda i: (0, 0)),
        ],
        out_shape=[
            jax.ShapeDtypeStruct((n, d), F32),
            jax.ShapeDtypeStruct((2, d), F32),
        ],
    )(o, wt, b, h)


def _combine_mlp_stats(t1, st1, t2, st2, g1, c1, g2, c2, m1t, mb1, m2t, mb2):
    n, d = t1.shape
    dmid = m1t.shape[1]
    br = _pick_block(n, cap=1000)
    ng = n // br
    nf = float(n)

    def body(t1_ref, s1_ref, t2_ref, s2_ref, g1_ref, c1_ref, g2_ref, c2_ref,
             w1_ref, b1_ref, w2_ref, b2_ref, t3_ref, st_ref):
        i = pl.program_id(0)
        mu1 = s1_ref[0:1, :] / nf
        va1 = s1_ref[1:2, :] / nf - mu1 * mu1
        h1 = g1_ref[...] * (t1_ref[...] - mu1) / jnp.sqrt(va1 + 1e-5) + c1_ref[...]
        mu2 = s2_ref[0:1, :] / nf
        va2 = s2_ref[1:2, :] / nf - mu2 * mu2
        h2 = g2_ref[...] * (t2_ref[...] - mu2) / jnp.sqrt(va2 + 1e-5) + c2_ref[...]
        op = h1 + h2
        z = jnp.maximum(jnp.dot(op, w1_ref[...], preferred_element_type=F32) + b1_ref[...], 0.0)
        t3 = op + jnp.dot(z, w2_ref[...], preferred_element_type=F32) + b2_ref[...]
        t3_ref[...] = t3

        @pl.when(i == 0)
        def _():
            st_ref[...] = jnp.zeros_like(st_ref)

        st_ref[0:1, :] += jnp.sum(t3, axis=0, keepdims=True)
        st_ref[1:2, :] += jnp.sum(t3 * t3, axis=0, keepdims=True)

    full = lambda shape: pl.BlockSpec(shape, lambda i: (0, 0))
    rows = pl.BlockSpec((br, d), lambda i: (i, 0))
    return pl.pallas_call(
        body,
        grid=(ng,),
        in_specs=[
            rows, full((2, d)), rows, full((2, d)),
            full((1, d)), full((1, d)), full((1, d)), full((1, d)),
            full((d, dmid)), full((1, dmid)), full((dmid, d)), full((1, d)),
        ],
        out_specs=[
            pl.BlockSpec((br, d), lambda i: (i, 0)),
            pl.BlockSpec((2, d), lambda i: (0, 0)),
        ],
        out_shape=[
            jax.ShapeDtypeStruct((n, d), F32),
            jax.ShapeDtypeStruct((2, d), F32),
        ],
    )(t1, st1, t2, st2, g1, c1, g2, c2, m1t, mb1, m2t, mb2)


def _final_bn_dec(t3, st3, g3, c3, decwt):
    n, d = t3.shape
    br = _pick_block(n, cap=1000)
    nf = float(n)

    def body(t_ref, s_ref, g_ref, c_ref, w_ref, o_ref, q_ref):
        mu = s_ref[0:1, :] / nf
        va = s_ref[1:2, :] / nf - mu * mu
        out = g_ref[...] * (t_ref[...] - mu) / jnp.sqrt(va + 1e-5) + c_ref[...]
        o_ref[...] = out
        q_ref[...] = jnp.dot(out, w_ref[...], preferred_element_type=F32)

    full = lambda shape: pl.BlockSpec(shape, lambda i: (0, 0))
    return pl.pallas_call(
        body,
        grid=(n // br,),
        in_specs=[
            pl.BlockSpec((br, d), lambda i: (i, 0)),
            full((2, d)), full((1, d)), full((1, d)), full((d, d)),
        ],
        out_specs=[
            pl.BlockSpec((br, d), lambda i: (i, 0)),
            pl.BlockSpec((br, d), lambda i: (i, 0)),
        ],
        out_shape=[
            jax.ShapeDtypeStruct((n, d), F32),
            jax.ShapeDtypeStruct((n, d), F32),
        ],
    )(t3, st3, g3, c3, decwt)


_NC = 2
_NS = 16
_NW = _NC * _NS


def _lane_gather(v, idx):
    dnums = lax.GatherDimensionNumbers(
        offset_dims=(), collapsed_slice_dims=(0,), start_index_map=(0,))
    return lax.gather(v, idx[:, None], dnums, (1,),
                      mode=lax.GatherScatterMode.PROMISE_IN_BOUNDS)


def _sc_message(src, dst, h, ee, zeros_init):
    e = src.shape[0]
    hd = h.shape[1]
    racc = zeros_init.shape[0]
    epw = e // _NW
    c_sz = 80
    nch = epw // c_sz
    rpt = racc // _NS
    mesh = plsc.VectorSubcoreMesh(core_axis_name="c", subcore_axis_name="s")

    @functools.partial(
        pl.kernel,
        out_type=jax.ShapeDtypeStruct((_NC * racc, hd), F32),
        mesh=mesh,
        scratch_types=[
            pltpu.VMEM((c_sz,), jnp.int32),
            pltpu.VMEM((c_sz,), jnp.int32),
            pltpu.VMEM((c_sz, hd), F32),
            pltpu.VMEM((c_sz, hd), F32),
            pltpu.VMEM_SHARED((racc, hd), F32),
            pltpu.SemaphoreType.DMA,
        ],
    )
    def k(src_hbm, dst_hbm, h_hbm, ee_hbm, z_hbm, out_hbm,
          src_v, dst_v, hrow_v, ee_v, acc_sh, sem):
        c = lax.axis_index("c")
        s = lax.axis_index("s")
        wid = c * _NS + s
        pltpu.sync_copy(z_hbm.at[pl.ds(s * rpt, rpt)], acc_sh.at[pl.ds(s * rpt, rpt)])
        plsc.subcore_barrier()

        def chunk(i, carry):
            base = wid * epw + i * c_sz
            pltpu.sync_copy(src_hbm.at[pl.ds(base, c_sz)], src_v)
            pltpu.sync_copy(dst_hbm.at[pl.ds(base, c_sz)], dst_v)
            pltpu.async_copy(h_hbm.at[src_v], hrow_v, sem).wait()
            pltpu.sync_copy(ee_hbm.at[pl.ds(base, c_sz)], ee_v)

            def row(r, carry2):
                for j in range(hd // 16):
                    sl = pl.ds(j * 16, 16)
                    hrow_v[r, sl] = jnp.maximum(hrow_v[r, sl] + ee_v[r, sl], 0.0)
                return carry2

            lax.fori_loop(0, c_sz, row, 0)
            pltpu.sync_copy(hrow_v, acc_sh.at[dst_v], add=True)
            return carry

        lax.fori_loop(0, nch, chunk, 0)
        plsc.subcore_barrier()
        pltpu.sync_copy(acc_sh.at[pl.ds(s * rpt, rpt)],
                        out_hbm.at[pl.ds(c * racc + s * rpt, rpt)])

    return k(src, dst, h, ee, zeros_init)


def _sc_decode(out3, q2, oi, di):
    pp = oi.shape[0]
    hd = out3.shape[1]
    ppw = pp // _NW
    cd = 128
    nch = ppw // cd
    mesh = plsc.VectorSubcoreMesh(core_axis_name="c", subcore_axis_name="s")

    @functools.partial(
        pl.kernel,
        out_type=jax.ShapeDtypeStruct((pp,), F32),
        mesh=mesh,
        scratch_types=[
            pltpu.VMEM((cd,), jnp.int32),
            pltpu.VMEM((cd,), jnp.int32),
            pltpu.VMEM((cd, hd), F32),
            pltpu.VMEM((cd, hd), F32),
            pltpu.VMEM((cd,), F32),
            pltpu.SemaphoreType.DMA,
        ],
    )
    def k(o_hbm, q_hbm, oi_hbm, di_hbm, res_hbm, oi_v, di_v, oe_v, de_v, res_v, sem):
        c = lax.axis_index("c")
        s = lax.axis_index("s")
        wid = c * _NS + s
        lane = lax.broadcasted_iota(jnp.int32, (16,), 0)

        def chunk(i, carry):
            base = wid * ppw + i * cd
            pltpu.sync_copy(oi_hbm.at[pl.ds(base, cd)], oi_v)
            pltpu.sync_copy(di_hbm.at[pl.ds(base, cd)], di_v)
            pltpu.async_copy(o_hbm.at[oi_v], oe_v, sem).wait()
            pltpu.async_copy(q_hbm.at[di_v], de_v, sem).wait()

            def grp(g, carry2):
                vec = jnp.zeros((16,), F32)
                for jj in range(16):
                    r = g * 16 + jj
                    acc = jnp.zeros((16,), F32)
                    for j in range(hd // 16):
                        sl = pl.ds(j * 16, 16)
                        acc = acc + oe_v[r, sl] * de_v[r, sl]
                    for kk in (1, 2, 4, 8):
                        acc = acc + _lane_gather(acc, lane ^ kk)
                    vec = jnp.where(lane == jj, acc, vec)
                res_v[pl.ds(g * 16, 16)] = vec
                return carry2

            lax.fori_loop(0, cd // 16, grp, 0)
            pltpu.sync_copy(res_v, res_hbm.at[pl.ds(base, cd)])
            return carry

        lax.fori_loop(0, nch, chunk, 0)

    return k(out3, q2, oi, di)


def kernel(x, edge_attr, params, edge_index, origin_idx, dest_idx):
    p = params
    n, idim = x.shape
    e = edge_attr.shape[0]
    hd = p["np2_W"].shape[0]
    nh = 4
    dh = hd // nh
    npairs = origin_idx.shape[0]

    r2 = lambda v: v.reshape(1, -1)

    h = _mlp2(x, p["np1_W"].T, r2(p["np1_b"]), p["np2_W"].T, r2(p["np2_b"]))
    ee = _mlp2(edge_attr, p["ep1_W"].T, r2(p["ep1_b"]), p["ep2_W"].T, r2(p["ep2_b"]))

    racc = 10240
    zinit = jnp.zeros((racc, hd), F32)
    parts = _sc_message(edge_index[0], edge_index[1], h, ee, zinit)
    a0 = lax.slice(parts, (0, 0), (n, hd))
    a1 = lax.slice(parts, (racc, 0), (racc + n, hd))

    t1, st1 = _gin_res_stats(h, a0, a1, p["gin1_W"].T, r2(p["gin1_b"]),
                             p["gin2_W"].T, r2(p["gin2_b"]))

    win = p["attn_in_W"]
    bin_ = p["attn_in_b"]
    wpad = jnp.zeros((hd, 3 * nh * 128), F32)
    bpad = jnp.zeros((3 * nh * 128,), F32)
    for part in range(3):
        for hh in range(nh):
            src_lo = part * hd + hh * dh
            dst_lo = (part * nh + hh) * 128
            wpad = wpad.at[:, dst_lo:dst_lo + dh].set(win[src_lo:src_lo + dh, :].T)
            bpad = bpad.at[dst_lo:dst_lo + dh].set(bin_[src_lo:src_lo + dh])
    qkv = _matmul_bias(h, wpad, r2(bpad))

    o_all = _attn_direct(qkv, nh, dh)

    wo = p["attn_out_W"]
    wo_pad = jnp.zeros((nh * 128, hd), F32)
    for hh in range(nh):
        wo_pad = wo_pad.at[hh * 128:hh * 128 + dh, :].set(wo[:, hh * dh:(hh + 1) * dh].T)
    t2, st2 = _lin_res_stats(o_all, wo_pad, r2(p["attn_out_b"]), h)

    t3, st3 = _combine_mlp_stats(
        t1, st1, t2, st2,
        r2(p["n1_g"]), r2(p["n1_b"]), r2(p["n2_g"]), r2(p["n2_b"]),
        p["mlp1_W"].T, r2(p["mlp1_b"]), p["mlp2_W"].T, r2(p["mlp2_b"]))

    out3, q2 = _final_bn_dec(t3, st3, r2(p["n3_g"]), r2(p["n3_b"]), p["dec_W"].T)

    ppad = ((npairs + 4096 - 1) // 4096) * 4096
    oi = jnp.pad(origin_idx, (0, ppad - npairs))
    di = jnp.pad(dest_idx, (0, ppad - npairs))
    res = _sc_decode(out3, q2, oi, di)
    return lax.slice(res, (0,), (npairs,))
